# Initial kernel scaffold; baseline (speedup 1.0000x reference)
import jax
import jax.numpy as jnp
from jax.experimental import pallas as pl


def kernel(x, edge_index, batch, W1, b1, W2, b2, W3, b3):
    raise NotImplementedError("write your pallas kernel here")


# speedup vs baseline: 22.0760x; 22.0760x over previous
import functools

import jax
import jax.numpy as jnp
from jax import lax
from jax.experimental import pallas as pl
from jax.experimental.pallas import tpu as pltpu
from jax.experimental.pallas import tpu_sc as plsc

N_NODES = 10000
D_HID = 64
NC = 2
NS = 16
RPT = N_NODES // NS
CHUNK = 400
ROWS_TC = 1000


def _sc_mesh():
    return plsc.VectorSubcoreMesh(core_axis_name="c", subcore_axis_name="s")


def _deg_body(col_hbm, out_hbm, col_v, ones_v, zbuf, acc_sh, *, ept):
    c = lax.axis_index("c")
    s = lax.axis_index("s")
    wid = c * NS + s
    one16 = jnp.ones((16,), jnp.float32)
    zero16 = jnp.zeros((16,), jnp.float32)

    def fill(i, _):
        ones_v[i, pl.ds(0, 16)] = one16
        zbuf[i, pl.ds(0, 16)] = zero16
        return 0
    lax.fori_loop(0, CHUNK, fill, 0)

    def fillz(i, _):
        zbuf[i, pl.ds(0, 16)] = zero16
        return 0
    lax.fori_loop(CHUNK, RPT, fillz, 0)

    pltpu.sync_copy(zbuf, acc_sh.at[pl.ds(s * RPT, RPT)])
    plsc.subcore_barrier()

    def body(i, _):
        base = wid * ept + i * CHUNK
        pltpu.sync_copy(col_hbm.at[pl.ds(base, CHUNK)], col_v)
        pltpu.sync_copy(ones_v, acc_sh.at[col_v], add=True)
        return 0
    lax.fori_loop(0, ept // CHUNK, body, 0)

    plsc.subcore_barrier()
    pltpu.sync_copy(acc_sh.at[pl.ds(s * RPT, RPT)], out_hbm.at[c, s])


def _make_deg_call(E):
    ept = E // (NC * NS)
    return functools.partial(
        pl.kernel,
        mesh=_sc_mesh(),
        compiler_params=pltpu.CompilerParams(use_tc_tiling_on_sc=False),
        out_type=jax.ShapeDtypeStruct((NC, NS, RPT, 16), jnp.float32),
        scratch_types=[
            pltpu.VMEM((CHUNK,), jnp.int32),
            pltpu.VMEM((CHUNK, 16), jnp.float32),
            pltpu.VMEM((RPT, 16), jnp.float32),
            pltpu.VMEM_SHARED((N_NODES, 16), jnp.float32),
        ],
    )(functools.partial(_deg_body, ept=ept))


def _scatter_body(hp_hbm, row_hbm, col_hbm, out_hbm,
                  row_v, col_v, rows_v, acc_sh, sem, *, ept):
    c = lax.axis_index("c")
    s = lax.axis_index("s")
    wid = c * NS + s
    zero16 = jnp.zeros((16,), jnp.float32)

    def fillz(i, _):
        for j in range(D_HID // 16):
            rows_v[i, pl.ds(j * 16, 16)] = zero16
        return 0
    lax.fori_loop(0, CHUNK, fillz, 0)

    pltpu.sync_copy(rows_v, acc_sh.at[pl.ds(s * RPT, CHUNK)])
    pltpu.sync_copy(rows_v.at[pl.ds(0, RPT - CHUNK)],
                    acc_sh.at[pl.ds(s * RPT + CHUNK, RPT - CHUNK)])
    plsc.subcore_barrier()

    def body(i, _):
        base = wid * ept + i * CHUNK
        pltpu.sync_copy(row_hbm.at[pl.ds(base, CHUNK)], row_v)
        pltpu.sync_copy(col_hbm.at[pl.ds(base, CHUNK)], col_v)
        pltpu.async_copy(hp_hbm.at[row_v], rows_v, sem).wait()
        pltpu.sync_copy(rows_v, acc_sh.at[col_v], add=True)
        return 0
    lax.fori_loop(0, ept // CHUNK, body, 0)

    plsc.subcore_barrier()
    pltpu.sync_copy(acc_sh.at[pl.ds(s * RPT, RPT)], out_hbm.at[c, s])


def _make_scatter_call(E):
    ept = E // (NC * NS)
    return functools.partial(
        pl.kernel,
        mesh=_sc_mesh(),
        compiler_params=pltpu.CompilerParams(use_tc_tiling_on_sc=False),
        out_type=jax.ShapeDtypeStruct((NC, NS, RPT, D_HID), jnp.float32),
        scratch_types=[
            pltpu.VMEM((CHUNK,), jnp.int32),
            pltpu.VMEM((CHUNK,), jnp.int32),
            pltpu.VMEM((CHUNK, D_HID), jnp.float32),
            pltpu.VMEM_SHARED((N_NODES, D_HID), jnp.float32),
            pltpu.SemaphoreType.DMA,
        ],
    )(functools.partial(_scatter_body, ept=ept))


def _tc1_body(d0_ref, d1_ref, x_ref, w_ref, hp_ref, dinv_ref):
    deg = d0_ref[:, 0:1] + d1_ref[:, 0:1] + 1.0
    dinv = lax.rsqrt(deg)
    dinv_ref[...] = dinv
    hp_ref[...] = dinv * jnp.dot(x_ref[...], w_ref[...],
                                 preferred_element_type=jnp.float32,
                                 precision=lax.Precision.HIGHEST)


def _tc1_call(d0, d1, x, w1t):
    n, din = x.shape
    grid = n // ROWS_TC
    return pl.pallas_call(
        _tc1_body,
        grid=(grid,),
        in_specs=[
            pl.BlockSpec((ROWS_TC, 16), lambda i: (i, 0)),
            pl.BlockSpec((ROWS_TC, 16), lambda i: (i, 0)),
            pl.BlockSpec((ROWS_TC, din), lambda i: (i, 0)),
            pl.BlockSpec((din, D_HID), lambda i: (0, 0)),
        ],
        out_specs=[
            pl.BlockSpec((ROWS_TC, D_HID), lambda i: (i, 0)),
            pl.BlockSpec((ROWS_TC, 1), lambda i: (i, 0)),
        ],
        out_shape=[
            jax.ShapeDtypeStruct((n, D_HID), jnp.float32),
            jax.ShapeDtypeStruct((n, 1), jnp.float32),
        ],
    )(d0, d1, x, w1t)


def _tc_mid_body(a0_ref, a1_ref, hp_ref, dinv_ref, b_ref, w_ref, out_ref):
    agg = dinv_ref[...] * (a0_ref[...] + a1_ref[...] + hp_ref[...]) + b_ref[...]
    h = jnp.where(agg >= 0, agg, 0.01 * agg)
    out_ref[...] = dinv_ref[...] * jnp.dot(h, w_ref[...],
                                           preferred_element_type=jnp.float32,
                                           precision=lax.Precision.HIGHEST)


def _tc_mid_call(a0, a1, hp, dinv, b, wt):
    n = hp.shape[0]
    grid = n // ROWS_TC
    return pl.pallas_call(
        _tc_mid_body,
        grid=(grid,),
        in_specs=[
            pl.BlockSpec((ROWS_TC, D_HID), lambda i: (i, 0)),
            pl.BlockSpec((ROWS_TC, D_HID), lambda i: (i, 0)),
            pl.BlockSpec((ROWS_TC, D_HID), lambda i: (i, 0)),
            pl.BlockSpec((ROWS_TC, 1), lambda i: (i, 0)),
            pl.BlockSpec((1, D_HID), lambda i: (0, 0)),
            pl.BlockSpec((D_HID, D_HID), lambda i: (0, 0)),
        ],
        out_specs=pl.BlockSpec((ROWS_TC, D_HID), lambda i: (i, 0)),
        out_shape=jax.ShapeDtypeStruct((n, D_HID), jnp.float32),
    )(a0, a1, hp, dinv, b, wt)


def _tc_fin_body(a0_ref, a1_ref, hp_ref, dinv_ref, b_ref, out_ref):
    agg = dinv_ref[...] * (a0_ref[...] + a1_ref[...] + hp_ref[...]) + b_ref[...]
    out_ref[...] = jnp.where(agg >= 0, agg, 0.01 * agg)


def _tc_fin_call(a0, a1, hp, dinv, b):
    n = hp.shape[0]
    grid = n // ROWS_TC
    return pl.pallas_call(
        _tc_fin_body,
        grid=(grid,),
        in_specs=[
            pl.BlockSpec((ROWS_TC, D_HID), lambda i: (i, 0)),
            pl.BlockSpec((ROWS_TC, D_HID), lambda i: (i, 0)),
            pl.BlockSpec((ROWS_TC, D_HID), lambda i: (i, 0)),
            pl.BlockSpec((ROWS_TC, 1), lambda i: (i, 0)),
            pl.BlockSpec((1, D_HID), lambda i: (0, 0)),
        ],
        out_specs=pl.BlockSpec((ROWS_TC, D_HID), lambda i: (i, 0)),
        out_shape=jax.ShapeDtypeStruct((n, D_HID), jnp.float32),
    )(a0, a1, hp, dinv, b)


def kernel(x, edge_index, batch, W1, b1, W2, b2, W3, b3):
    E = edge_index.shape[1]
    row = edge_index[0]
    col = edge_index[1]

    degp = _make_deg_call(E)(col).reshape(NC, N_NODES, 16)
    hp1, dinv = _tc1_call(degp[0], degp[1], x, W1.T)

    scat = _make_scatter_call(E)

    def run_scat(hp):
        return scat(hp, row, col).reshape(NC, N_NODES, D_HID)

    acc = run_scat(hp1)
    hp2 = _tc_mid_call(acc[0], acc[1], hp1, dinv, b1.reshape(1, -1), W2.T)
    acc = run_scat(hp2)
    hp3 = _tc_mid_call(acc[0], acc[1], hp2, dinv, b2.reshape(1, -1), W3.T)
    acc = run_scat(hp3)
    return _tc_fin_call(acc[0], acc[1], hp3, dinv, b3.reshape(1, -1))


# speedup vs baseline: 28.9161x; 1.3098x over previous
import functools

import jax
import jax.numpy as jnp
from jax import lax
from jax.experimental import pallas as pl
from jax.experimental.pallas import tpu as pltpu
from jax.experimental.pallas import tpu_sc as plsc

N_NODES = 10000
D_HID = 64
NC = 2
NS = 16
RPT = N_NODES // NS
CHUNK = 400
ROWS_TC = 1000


def _sc_mesh():
    return plsc.VectorSubcoreMesh(core_axis_name="c", subcore_axis_name="s")


DCH = 2000


def _copy_idx_local(src_v, dst_v, n):
    for j in range(n // 16):
        dst_v[pl.ds(j * 16, 16)] = src_v[pl.ds(j * 16, 16)]


def _deg_body(col_hbm, out_hbm, col_v0, col_v1, scol_v0, scol_v1,
              ones_v, zbuf, acc_sh, sem_i0, sem_i1, sem_s0, sem_s1, *, ept):
    c = lax.axis_index("c")
    s = lax.axis_index("s")
    wid = c * NS + s
    ebase = wid * ept
    one16 = jnp.ones((16,), jnp.float32)
    zero16 = jnp.zeros((16,), jnp.float32)

    def fill(i, _):
        ones_v[i, pl.ds(0, 16)] = one16
        return 0
    lax.fori_loop(0, DCH, fill, 0)

    def fillz(i, _):
        zbuf[i, pl.ds(0, 16)] = zero16
        return 0
    lax.fori_loop(0, RPT, fillz, 0)

    pltpu.sync_copy(zbuf, acc_sh.at[pl.ds(s * RPT, RPT)])

    nch = ept // DCH
    pltpu.async_copy(col_hbm.at[pl.ds(ebase, DCH)], col_v0, sem_i0)
    pltpu.async_copy(col_hbm.at[pl.ds(ebase + DCH, DCH)], col_v1, sem_i1)
    plsc.subcore_barrier()

    def half(g, col_v, scol_v, sem_i, sem_s, first):
        @pl.when(jnp.logical_not(first))
        def _():
            pltpu.make_async_copy(ones_v, acc_sh.at[scol_v], sem_s).wait()
        pltpu.make_async_copy(col_hbm.at[pl.ds(ebase + g * DCH, DCH)],
                              col_v, sem_i).wait()
        _copy_idx_local(col_v, scol_v, DCH)
        sc = pltpu.make_async_copy(ones_v, acc_sh.at[scol_v], sem_s)
        sc.start(add=True)
        @pl.when(g + 2 < nch)
        def _():
            pltpu.async_copy(col_hbm.at[pl.ds(ebase + (g + 2) * DCH, DCH)],
                             col_v, sem_i)

    def body(g2, _):
        half(g2 * 2, col_v0, scol_v0, sem_i0, sem_s0, g2 == 0)
        half(g2 * 2 + 1, col_v1, scol_v1, sem_i1, sem_s1, g2 == 0)
        return 0
    lax.fori_loop(0, nch // 2, body, 0)

    if nch % 2 == 1:
        half(nch - 1, col_v0, scol_v0, sem_i0, sem_s0, False)

    pltpu.make_async_copy(ones_v, acc_sh.at[scol_v0], sem_s0).wait()
    pltpu.make_async_copy(ones_v, acc_sh.at[scol_v1], sem_s1).wait()
    plsc.subcore_barrier()
    pltpu.sync_copy(acc_sh.at[pl.ds(s * RPT, RPT)], out_hbm.at[c, s])


def _make_deg_call(E):
    ept = E // (NC * NS)
    return functools.partial(
        pl.kernel,
        mesh=_sc_mesh(),
        compiler_params=pltpu.CompilerParams(use_tc_tiling_on_sc=False),
        out_type=jax.ShapeDtypeStruct((NC, NS, RPT, 16), jnp.float32),
        scratch_types=[
            pltpu.VMEM((DCH,), jnp.int32),
            pltpu.VMEM((DCH,), jnp.int32),
            pltpu.VMEM((DCH,), jnp.int32),
            pltpu.VMEM((DCH,), jnp.int32),
            pltpu.VMEM((DCH, 16), jnp.float32),
            pltpu.VMEM((RPT, 16), jnp.float32),
            pltpu.VMEM_SHARED((N_NODES, 16), jnp.float32),
            pltpu.SemaphoreType.DMA,
            pltpu.SemaphoreType.DMA,
            pltpu.SemaphoreType.DMA,
            pltpu.SemaphoreType.DMA,
        ],
    )(functools.partial(_deg_body, ept=ept))


def _scatter_body(hp_hbm, row_hbm, col_hbm, out_hbm,
                  row_v0, row_v1, col_v0, col_v1, scol_v0, scol_v1,
                  rows_v0, rows_v1, acc_sh,
                  sem_i0, sem_i1, sem_g, sem_s0, sem_s1, *, ept):
    c = lax.axis_index("c")
    s = lax.axis_index("s")
    wid = c * NS + s
    ebase = wid * ept
    zero16 = jnp.zeros((16,), jnp.float32)

    def fillz(i, _):
        for j in range(D_HID // 16):
            rows_v0[i, pl.ds(j * 16, 16)] = zero16
        return 0
    lax.fori_loop(0, CHUNK, fillz, 0)

    pltpu.sync_copy(rows_v0, acc_sh.at[pl.ds(s * RPT, CHUNK)])
    pltpu.sync_copy(rows_v0.at[pl.ds(0, RPT - CHUNK)],
                    acc_sh.at[pl.ds(s * RPT + CHUNK, RPT - CHUNK)])

    nch = ept // CHUNK
    pltpu.async_copy(row_hbm.at[pl.ds(ebase, CHUNK)], row_v0, sem_i0)
    pltpu.async_copy(col_hbm.at[pl.ds(ebase, CHUNK)], col_v0, sem_i0)
    pltpu.async_copy(row_hbm.at[pl.ds(ebase + CHUNK, CHUNK)], row_v1, sem_i1)
    pltpu.async_copy(col_hbm.at[pl.ds(ebase + CHUNK, CHUNK)], col_v1, sem_i1)
    plsc.subcore_barrier()

    def half(g, row_v, col_v, scol_v, rows_v, sem_i, sem_s, first):
        @pl.when(jnp.logical_not(first))
        def _():
            pltpu.make_async_copy(rows_v, acc_sh.at[scol_v], sem_s).wait()
        pltpu.make_async_copy(row_hbm.at[pl.ds(ebase + g * CHUNK, CHUNK)],
                              row_v, sem_i).wait()
        pltpu.make_async_copy(col_hbm.at[pl.ds(ebase + g * CHUNK, CHUNK)],
                              col_v, sem_i).wait()
        pltpu.async_copy(hp_hbm.at[row_v], rows_v, sem_g).wait()
        _copy_idx_local(col_v, scol_v, CHUNK)
        sc = pltpu.make_async_copy(rows_v, acc_sh.at[scol_v], sem_s)
        sc.start(add=True)
        @pl.when(g + 2 < nch)
        def _():
            nb = ebase + (g + 2) * CHUNK
            pltpu.async_copy(row_hbm.at[pl.ds(nb, CHUNK)], row_v, sem_i)
            pltpu.async_copy(col_hbm.at[pl.ds(nb, CHUNK)], col_v, sem_i)

    def body(g2, _):
        half(g2 * 2, row_v0, col_v0, scol_v0, rows_v0, sem_i0, sem_s0, g2 == 0)
        half(g2 * 2 + 1, row_v1, col_v1, scol_v1, rows_v1, sem_i1, sem_s1,
             g2 == 0)
        return 0
    lax.fori_loop(0, nch // 2, body, 0)

    if nch % 2 == 1:
        half(nch - 1, row_v0, col_v0, scol_v0, rows_v0, sem_i0, sem_s0, False)

    pltpu.make_async_copy(rows_v0, acc_sh.at[scol_v0], sem_s0).wait()
    pltpu.make_async_copy(rows_v1, acc_sh.at[scol_v1], sem_s1).wait()
    plsc.subcore_barrier()
    pltpu.sync_copy(acc_sh.at[pl.ds(s * RPT, RPT)], out_hbm.at[c, s])


def _make_scatter_call(E):
    ept = E // (NC * NS)
    return functools.partial(
        pl.kernel,
        mesh=_sc_mesh(),
        compiler_params=pltpu.CompilerParams(use_tc_tiling_on_sc=False),
        out_type=jax.ShapeDtypeStruct((NC, NS, RPT, D_HID), jnp.float32),
        scratch_types=[
            pltpu.VMEM((CHUNK,), jnp.int32),
            pltpu.VMEM((CHUNK,), jnp.int32),
            pltpu.VMEM((CHUNK,), jnp.int32),
            pltpu.VMEM((CHUNK,), jnp.int32),
            pltpu.VMEM((CHUNK,), jnp.int32),
            pltpu.VMEM((CHUNK,), jnp.int32),
            pltpu.VMEM((CHUNK, D_HID), jnp.float32),
            pltpu.VMEM((CHUNK, D_HID), jnp.float32),
            pltpu.VMEM_SHARED((N_NODES, D_HID), jnp.float32),
            pltpu.SemaphoreType.DMA,
            pltpu.SemaphoreType.DMA,
            pltpu.SemaphoreType.DMA,
            pltpu.SemaphoreType.DMA,
            pltpu.SemaphoreType.DMA,
        ],
    )(functools.partial(_scatter_body, ept=ept))


def _tc1_body(d0_ref, d1_ref, x_ref, w_ref, hp_ref, dinv_ref):
    deg = d0_ref[:, 0:1] + d1_ref[:, 0:1] + 1.0
    dinv = lax.rsqrt(deg)
    dinv_ref[...] = dinv
    hp_ref[...] = dinv * jnp.dot(x_ref[...], w_ref[...],
                                 preferred_element_type=jnp.float32,
                                 precision=lax.Precision.HIGHEST)


def _tc1_call(d0, d1, x, w1t):
    n, din = x.shape
    grid = n // ROWS_TC
    return pl.pallas_call(
        _tc1_body,
        grid=(grid,),
        in_specs=[
            pl.BlockSpec((ROWS_TC, 16), lambda i: (i, 0)),
            pl.BlockSpec((ROWS_TC, 16), lambda i: (i, 0)),
            pl.BlockSpec((ROWS_TC, din), lambda i: (i, 0)),
            pl.BlockSpec((din, D_HID), lambda i: (0, 0)),
        ],
        out_specs=[
            pl.BlockSpec((ROWS_TC, D_HID), lambda i: (i, 0)),
            pl.BlockSpec((ROWS_TC, 1), lambda i: (i, 0)),
        ],
        out_shape=[
            jax.ShapeDtypeStruct((n, D_HID), jnp.float32),
            jax.ShapeDtypeStruct((n, 1), jnp.float32),
        ],
    )(d0, d1, x, w1t)


def _tc_mid_body(a0_ref, a1_ref, hp_ref, dinv_ref, b_ref, w_ref, out_ref):
    agg = dinv_ref[...] * (a0_ref[...] + a1_ref[...] + hp_ref[...]) + b_ref[...]
    h = jnp.where(agg >= 0, agg, 0.01 * agg)
    out_ref[...] = dinv_ref[...] * jnp.dot(h, w_ref[...],
                                           preferred_element_type=jnp.float32,
                                           precision=lax.Precision.HIGHEST)


def _tc_mid_call(a0, a1, hp, dinv, b, wt):
    n = hp.shape[0]
    grid = n // ROWS_TC
    return pl.pallas_call(
        _tc_mid_body,
        grid=(grid,),
        in_specs=[
            pl.BlockSpec((ROWS_TC, D_HID), lambda i: (i, 0)),
            pl.BlockSpec((ROWS_TC, D_HID), lambda i: (i, 0)),
            pl.BlockSpec((ROWS_TC, D_HID), lambda i: (i, 0)),
            pl.BlockSpec((ROWS_TC, 1), lambda i: (i, 0)),
            pl.BlockSpec((1, D_HID), lambda i: (0, 0)),
            pl.BlockSpec((D_HID, D_HID), lambda i: (0, 0)),
        ],
        out_specs=pl.BlockSpec((ROWS_TC, D_HID), lambda i: (i, 0)),
        out_shape=jax.ShapeDtypeStruct((n, D_HID), jnp.float32),
    )(a0, a1, hp, dinv, b, wt)


def _tc_fin_body(a0_ref, a1_ref, hp_ref, dinv_ref, b_ref, out_ref):
    agg = dinv_ref[...] * (a0_ref[...] + a1_ref[...] + hp_ref[...]) + b_ref[...]
    out_ref[...] = jnp.where(agg >= 0, agg, 0.01 * agg)


def _tc_fin_call(a0, a1, hp, dinv, b):
    n = hp.shape[0]
    grid = n // ROWS_TC
    return pl.pallas_call(
        _tc_fin_body,
        grid=(grid,),
        in_specs=[
            pl.BlockSpec((ROWS_TC, D_HID), lambda i: (i, 0)),
            pl.BlockSpec((ROWS_TC, D_HID), lambda i: (i, 0)),
            pl.BlockSpec((ROWS_TC, D_HID), lambda i: (i, 0)),
            pl.BlockSpec((ROWS_TC, 1), lambda i: (i, 0)),
            pl.BlockSpec((1, D_HID), lambda i: (0, 0)),
        ],
        out_specs=pl.BlockSpec((ROWS_TC, D_HID), lambda i: (i, 0)),
        out_shape=jax.ShapeDtypeStruct((n, D_HID), jnp.float32),
    )(a0, a1, hp, dinv, b)


def kernel(x, edge_index, batch, W1, b1, W2, b2, W3, b3):
    E = edge_index.shape[1]
    row = edge_index[0]
    col = edge_index[1]

    degp = _make_deg_call(E)(col).reshape(NC, N_NODES, 16)
    hp1, dinv = _tc1_call(degp[0], degp[1], x, W1.T)

    scat = _make_scatter_call(E)

    def run_scat(hp):
        return scat(hp, row, col).reshape(NC, N_NODES, D_HID)

    acc = run_scat(hp1)
    hp2 = _tc_mid_call(acc[0], acc[1], hp1, dinv, b1.reshape(1, -1), W2.T)
    acc = run_scat(hp2)
    hp3 = _tc_mid_call(acc[0], acc[1], hp2, dinv, b2.reshape(1, -1), W3.T)
    acc = run_scat(hp3)
    return _tc_fin_call(acc[0], acc[1], hp3, dinv, b3.reshape(1, -1))


# speedup vs baseline: 29.6962x; 1.0270x over previous
import functools

import jax
import jax.numpy as jnp
from jax import lax
from jax.experimental import pallas as pl
from jax.experimental.pallas import tpu as pltpu
from jax.experimental.pallas import tpu_sc as plsc

N_NODES = 10000
D_HID = 64
NC = 2
NS = 16
RPT = N_NODES // NS
CHUNK = 400
ROWS_TC = 1000


def _sc_mesh():
    return plsc.VectorSubcoreMesh(core_axis_name="c", subcore_axis_name="s")


DCH = 2000


def _copy_idx_local(src_v, dst_v, n):
    for j in range(n // 16):
        dst_v[pl.ds(j * 16, 16)] = src_v[pl.ds(j * 16, 16)]


def _deg_body(col_hbm, out_hbm, col_v0, col_v1, scol_v0, scol_v1,
              ones_v, zbuf, acc_sh, sem_i0, sem_i1, sem_s0, sem_s1, *, ept):
    c = lax.axis_index("c")
    s = lax.axis_index("s")
    wid = c * NS + s
    ebase = wid * ept
    one16 = jnp.ones((16,), jnp.float32)
    zero16 = jnp.zeros((16,), jnp.float32)

    def fill(i, _):
        ones_v[i, pl.ds(0, 16)] = one16
        return 0
    lax.fori_loop(0, DCH, fill, 0)

    def fillz(i, _):
        zbuf[i, pl.ds(0, 16)] = zero16
        return 0
    lax.fori_loop(0, RPT, fillz, 0)

    pltpu.sync_copy(zbuf, acc_sh.at[pl.ds(s * RPT, RPT)])

    nch = ept // DCH
    pltpu.async_copy(col_hbm.at[pl.ds(ebase, DCH)], col_v0, sem_i0)
    pltpu.async_copy(col_hbm.at[pl.ds(ebase + DCH, DCH)], col_v1, sem_i1)
    plsc.subcore_barrier()

    def half(g, col_v, scol_v, sem_i, sem_s, first):
        @pl.when(jnp.logical_not(first))
        def _():
            pltpu.make_async_copy(ones_v, acc_sh.at[scol_v], sem_s).wait()
        pltpu.make_async_copy(col_hbm.at[pl.ds(ebase + g * DCH, DCH)],
                              col_v, sem_i).wait()
        _copy_idx_local(col_v, scol_v, DCH)
        sc = pltpu.make_async_copy(ones_v, acc_sh.at[scol_v], sem_s)
        sc.start(add=True)
        @pl.when(g + 2 < nch)
        def _():
            pltpu.async_copy(col_hbm.at[pl.ds(ebase + (g + 2) * DCH, DCH)],
                             col_v, sem_i)

    def body(g2, _):
        half(g2 * 2, col_v0, scol_v0, sem_i0, sem_s0, g2 == 0)
        half(g2 * 2 + 1, col_v1, scol_v1, sem_i1, sem_s1, g2 == 0)
        return 0
    lax.fori_loop(0, nch // 2, body, 0)

    if nch % 2 == 1:
        half(nch - 1, col_v0, scol_v0, sem_i0, sem_s0, False)

    pltpu.make_async_copy(ones_v, acc_sh.at[scol_v0], sem_s0).wait()
    pltpu.make_async_copy(ones_v, acc_sh.at[scol_v1], sem_s1).wait()
    plsc.subcore_barrier()
    pltpu.sync_copy(acc_sh.at[pl.ds(s * RPT, RPT)], out_hbm.at[c, s])


def _make_deg_call(E):
    ept = E // (NC * NS)
    return functools.partial(
        pl.kernel,
        mesh=_sc_mesh(),
        compiler_params=pltpu.CompilerParams(use_tc_tiling_on_sc=False),
        out_type=jax.ShapeDtypeStruct((NC, NS, RPT, 16), jnp.float32),
        scratch_types=[
            pltpu.VMEM((DCH,), jnp.int32),
            pltpu.VMEM((DCH,), jnp.int32),
            pltpu.VMEM((DCH,), jnp.int32),
            pltpu.VMEM((DCH,), jnp.int32),
            pltpu.VMEM((DCH, 16), jnp.float32),
            pltpu.VMEM((RPT, 16), jnp.float32),
            pltpu.VMEM_SHARED((N_NODES, 16), jnp.float32),
            pltpu.SemaphoreType.DMA,
            pltpu.SemaphoreType.DMA,
            pltpu.SemaphoreType.DMA,
            pltpu.SemaphoreType.DMA,
        ],
    )(functools.partial(_deg_body, ept=ept))


def _scatter_body(hp_hbm, row_hbm, col_hbm, out_hbm,
                  row_v0, row_v1, row_v2, col_v0, col_v1, col_v2,
                  scol_v0, scol_v1, scol_v2, rows_v0, rows_v1, rows_v2,
                  acc_sh,
                  sem_i0, sem_i1, sem_i2, sem_g0, sem_g1, sem_g2,
                  sem_s0, sem_s1, sem_s2, *, ept):
    c = lax.axis_index("c")
    s = lax.axis_index("s")
    wid = c * NS + s
    ebase = wid * ept
    zero16 = jnp.zeros((16,), jnp.float32)

    row_v = [row_v0, row_v1, row_v2]
    col_v = [col_v0, col_v1, col_v2]
    scol_v = [scol_v0, scol_v1, scol_v2]
    rows_v = [rows_v0, rows_v1, rows_v2]
    sem_i = [sem_i0, sem_i1, sem_i2]
    sem_g = [sem_g0, sem_g1, sem_g2]
    sem_s = [sem_s0, sem_s1, sem_s2]

    def fillz(i, _):
        for j in range(D_HID // 16):
            rows_v0[i, pl.ds(j * 16, 16)] = zero16
        return 0
    lax.fori_loop(0, CHUNK, fillz, 0)

    pltpu.sync_copy(rows_v0, acc_sh.at[pl.ds(s * RPT, CHUNK)])
    pltpu.sync_copy(rows_v0.at[pl.ds(0, RPT - CHUNK)],
                    acc_sh.at[pl.ds(s * RPT + CHUNK, RPT - CHUNK)])

    nch = ept // CHUNK
    for k in range(3):
        pltpu.async_copy(row_hbm.at[pl.ds(ebase + k * CHUNK, CHUNK)],
                         row_v[k], sem_i[k])
        pltpu.async_copy(col_hbm.at[pl.ds(ebase + k * CHUNK, CHUNK)],
                         col_v[k], sem_i[k])
    plsc.subcore_barrier()

    def start_gather(g, k):
        @pl.when(g >= 3)
        def _():
            pltpu.make_async_copy(rows_v[k], acc_sh.at[scol_v[k]],
                                  sem_s[k]).wait()
        pltpu.make_async_copy(row_hbm.at[pl.ds(ebase + g * CHUNK, CHUNK)],
                              row_v[k], sem_i[k]).wait()
        pltpu.make_async_copy(col_hbm.at[pl.ds(ebase + g * CHUNK, CHUNK)],
                              col_v[k], sem_i[k]).wait()
        pltpu.async_copy(hp_hbm.at[row_v[k]], rows_v[k], sem_g[k])

    def finish_prev(g, k1):
        pltpu.make_async_copy(hp_hbm.at[row_v[k1]], rows_v[k1],
                              sem_g[k1]).wait()
        _copy_idx_local(col_v[k1], scol_v[k1], CHUNK)
        pltpu.make_async_copy(rows_v[k1], acc_sh.at[scol_v[k1]],
                              sem_s[k1]).start(add=True)
        @pl.when(g + 2 < nch)
        def _():
            nb = ebase + (g + 2) * CHUNK
            pltpu.async_copy(row_hbm.at[pl.ds(nb, CHUNK)], row_v[k1],
                             sem_i[k1])
            pltpu.async_copy(col_hbm.at[pl.ds(nb, CHUNK)], col_v[k1],
                             sem_i[k1])

    def body(g3, _):
        for k in range(3):
            g = g3 * 3 + k
            start_gather(g, k)
            @pl.when(g > 0)
            def _():
                finish_prev(g, (k + 2) % 3)
        return 0
    lax.fori_loop(0, nch // 3, body, 0)

    for t in range(nch - (nch // 3) * 3):
        g = (nch // 3) * 3 + t
        k = g % 3
        start_gather(g, k)
        finish_prev(g, (k + 2) % 3)
    klast = (nch - 1) % 3
    pltpu.make_async_copy(hp_hbm.at[row_v[klast]], rows_v[klast],
                          sem_g[klast]).wait()
    _copy_idx_local(col_v[klast], scol_v[klast], CHUNK)
    pltpu.make_async_copy(rows_v[klast], acc_sh.at[scol_v[klast]],
                          sem_s[klast]).start(add=True)

    for k in range(3):
        pltpu.make_async_copy(rows_v[k], acc_sh.at[scol_v[k]], sem_s[k]).wait()
    plsc.subcore_barrier()
    pltpu.sync_copy(acc_sh.at[pl.ds(s * RPT, RPT)], out_hbm.at[c, s])


def _make_scatter_call(E):
    ept = E // (NC * NS)
    return functools.partial(
        pl.kernel,
        mesh=_sc_mesh(),
        compiler_params=pltpu.CompilerParams(use_tc_tiling_on_sc=False),
        out_type=jax.ShapeDtypeStruct((NC, NS, RPT, D_HID), jnp.float32),
        scratch_types=(
            [pltpu.VMEM((CHUNK,), jnp.int32)] * 6
            + [pltpu.VMEM((CHUNK,), jnp.int32)] * 3
            + [pltpu.VMEM((CHUNK, D_HID), jnp.float32)] * 3
            + [pltpu.VMEM_SHARED((N_NODES, D_HID), jnp.float32)]
            + [pltpu.SemaphoreType.DMA] * 9
        ),
    )(functools.partial(_scatter_body, ept=ept))


def _tc1_body(d0_ref, d1_ref, x_ref, w_ref, hp_ref, dinv_ref):
    deg = d0_ref[:, 0:1] + d1_ref[:, 0:1] + 1.0
    dinv = lax.rsqrt(deg)
    dinv_ref[...] = dinv
    hp_ref[...] = dinv * jnp.dot(x_ref[...], w_ref[...],
                                 preferred_element_type=jnp.float32,
                                 precision=lax.Precision.HIGHEST)


def _tc1_call(d0, d1, x, w1t):
    n, din = x.shape
    grid = n // ROWS_TC
    return pl.pallas_call(
        _tc1_body,
        grid=(grid,),
        in_specs=[
            pl.BlockSpec((ROWS_TC, 16), lambda i: (i, 0)),
            pl.BlockSpec((ROWS_TC, 16), lambda i: (i, 0)),
            pl.BlockSpec((ROWS_TC, din), lambda i: (i, 0)),
            pl.BlockSpec((din, D_HID), lambda i: (0, 0)),
        ],
        out_specs=[
            pl.BlockSpec((ROWS_TC, D_HID), lambda i: (i, 0)),
            pl.BlockSpec((ROWS_TC, 1), lambda i: (i, 0)),
        ],
        out_shape=[
            jax.ShapeDtypeStruct((n, D_HID), jnp.float32),
            jax.ShapeDtypeStruct((n, 1), jnp.float32),
        ],
    )(d0, d1, x, w1t)


def _tc_mid_body(a0_ref, a1_ref, hp_ref, dinv_ref, b_ref, w_ref, out_ref):
    agg = dinv_ref[...] * (a0_ref[...] + a1_ref[...] + hp_ref[...]) + b_ref[...]
    h = jnp.where(agg >= 0, agg, 0.01 * agg)
    out_ref[...] = dinv_ref[...] * jnp.dot(h, w_ref[...],
                                           preferred_element_type=jnp.float32,
                                           precision=lax.Precision.HIGHEST)


def _tc_mid_call(a0, a1, hp, dinv, b, wt):
    n = hp.shape[0]
    grid = n // ROWS_TC
    return pl.pallas_call(
        _tc_mid_body,
        grid=(grid,),
        in_specs=[
            pl.BlockSpec((ROWS_TC, D_HID), lambda i: (i, 0)),
            pl.BlockSpec((ROWS_TC, D_HID), lambda i: (i, 0)),
            pl.BlockSpec((ROWS_TC, D_HID), lambda i: (i, 0)),
            pl.BlockSpec((ROWS_TC, 1), lambda i: (i, 0)),
            pl.BlockSpec((1, D_HID), lambda i: (0, 0)),
            pl.BlockSpec((D_HID, D_HID), lambda i: (0, 0)),
        ],
        out_specs=pl.BlockSpec((ROWS_TC, D_HID), lambda i: (i, 0)),
        out_shape=jax.ShapeDtypeStruct((n, D_HID), jnp.float32),
    )(a0, a1, hp, dinv, b, wt)


def _tc_fin_body(a0_ref, a1_ref, hp_ref, dinv_ref, b_ref, out_ref):
    agg = dinv_ref[...] * (a0_ref[...] + a1_ref[...] + hp_ref[...]) + b_ref[...]
    out_ref[...] = jnp.where(agg >= 0, agg, 0.01 * agg)


def _tc_fin_call(a0, a1, hp, dinv, b):
    n = hp.shape[0]
    grid = n // ROWS_TC
    return pl.pallas_call(
        _tc_fin_body,
        grid=(grid,),
        in_specs=[
            pl.BlockSpec((ROWS_TC, D_HID), lambda i: (i, 0)),
            pl.BlockSpec((ROWS_TC, D_HID), lambda i: (i, 0)),
            pl.BlockSpec((ROWS_TC, D_HID), lambda i: (i, 0)),
            pl.BlockSpec((ROWS_TC, 1), lambda i: (i, 0)),
            pl.BlockSpec((1, D_HID), lambda i: (0, 0)),
        ],
        out_specs=pl.BlockSpec((ROWS_TC, D_HID), lambda i: (i, 0)),
        out_shape=jax.ShapeDtypeStruct((n, D_HID), jnp.float32),
    )(a0, a1, hp, dinv, b)


def kernel(x, edge_index, batch, W1, b1, W2, b2, W3, b3):
    E = edge_index.shape[1]
    row = edge_index[0]
    col = edge_index[1]

    degp = _make_deg_call(E)(col).reshape(NC, N_NODES, 16)
    hp1, dinv = _tc1_call(degp[0], degp[1], x, W1.T)

    scat = _make_scatter_call(E)

    def run_scat(hp):
        return scat(hp, row, col).reshape(NC, N_NODES, D_HID)

    acc = run_scat(hp1)
    hp2 = _tc_mid_call(acc[0], acc[1], hp1, dinv, b1.reshape(1, -1), W2.T)
    acc = run_scat(hp2)
    hp3 = _tc_mid_call(acc[0], acc[1], hp2, dinv, b2.reshape(1, -1), W3.T)
    acc = run_scat(hp3)
    return _tc_fin_call(acc[0], acc[1], hp3, dinv, b3.reshape(1, -1))


# speedup vs baseline: 36.6134x; 1.2329x over previous
import functools

import jax
import jax.numpy as jnp
from jax import lax
from jax.experimental import pallas as pl
from jax.experimental.pallas import tpu as pltpu
from jax.experimental.pallas import tpu_sc as plsc

N_NODES = 10000
D_HID = 64
NC = 2
NS = 16
RPT = N_NODES // NS
CHUNK = 400
DCH = 2000
ROWS_TC = 1000


def _sc_mesh():
    return plsc.VectorSubcoreMesh(core_axis_name="c", subcore_axis_name="s")


def _copy_idx_local(src_v, dst_v, n):
    for j in range(n // 16):
        dst_v[pl.ds(j * 16, 16)] = src_v[pl.ds(j * 16, 16)]


def _deg_body(ei_hbm, out_hbm, col_v0, col_v1, scol_v0, scol_v1,
              ones_v, zbuf, acc_sh, sem_i0, sem_i1, sem_s0, sem_s1, *, ept):
    c = lax.axis_index("c")
    s = lax.axis_index("s")
    wid = c * NS + s
    ebase = wid * ept
    one16 = jnp.ones((16,), jnp.float32)
    zero16 = jnp.zeros((16,), jnp.float32)

    def fill(i, _):
        ones_v[i, pl.ds(0, 16)] = one16
        return 0
    lax.fori_loop(0, DCH, fill, 0)

    def fillz(i, _):
        zbuf[i, pl.ds(0, 16)] = zero16
        return 0
    lax.fori_loop(0, RPT, fillz, 0)

    pltpu.sync_copy(zbuf, acc_sh.at[pl.ds(s * RPT, RPT)])

    nch = ept // DCH
    pltpu.async_copy(ei_hbm.at[1, pl.ds(ebase, DCH)], col_v0, sem_i0)
    pltpu.async_copy(ei_hbm.at[1, pl.ds(ebase + DCH, DCH)], col_v1, sem_i1)
    plsc.subcore_barrier()

    def half(g, col_v, scol_v, sem_i, sem_s, first):
        @pl.when(jnp.logical_not(first))
        def _():
            pltpu.make_async_copy(ones_v, acc_sh.at[scol_v], sem_s).wait()
        pltpu.make_async_copy(ei_hbm.at[1, pl.ds(ebase + g * DCH, DCH)],
                              col_v, sem_i).wait()
        _copy_idx_local(col_v, scol_v, DCH)
        sc = pltpu.make_async_copy(ones_v, acc_sh.at[scol_v], sem_s)
        sc.start(add=True)
        @pl.when(g + 2 < nch)
        def _():
            pltpu.async_copy(ei_hbm.at[1, pl.ds(ebase + (g + 2) * DCH, DCH)],
                             col_v, sem_i)

    def body(g2, _):
        half(g2 * 2, col_v0, scol_v0, sem_i0, sem_s0, g2 == 0)
        half(g2 * 2 + 1, col_v1, scol_v1, sem_i1, sem_s1, g2 == 0)
        return 0
    lax.fori_loop(0, nch // 2, body, 0)

    if nch % 2 == 1:
        half(nch - 1, col_v0, scol_v0, sem_i0, sem_s0, False)

    pltpu.make_async_copy(ones_v, acc_sh.at[scol_v0], sem_s0).wait()
    pltpu.make_async_copy(ones_v, acc_sh.at[scol_v1], sem_s1).wait()
    plsc.subcore_barrier()
    pltpu.sync_copy(acc_sh.at[pl.ds(s * RPT, RPT)],
                    out_hbm.at[c, pl.ds(s * RPT, RPT)])


def _make_deg_call(E):
    ept = E // (NC * NS)
    return functools.partial(
        pl.kernel,
        mesh=_sc_mesh(),
        compiler_params=pltpu.CompilerParams(use_tc_tiling_on_sc=False),
        out_type=jax.ShapeDtypeStruct((NC, N_NODES, 16), jnp.float32),
        scratch_types=[
            pltpu.VMEM((DCH,), jnp.int32),
            pltpu.VMEM((DCH,), jnp.int32),
            pltpu.VMEM((DCH,), jnp.int32),
            pltpu.VMEM((DCH,), jnp.int32),
            pltpu.VMEM((DCH, 16), jnp.float32),
            pltpu.VMEM((RPT, 16), jnp.float32),
            pltpu.VMEM_SHARED((N_NODES, 16), jnp.float32),
            pltpu.SemaphoreType.DMA,
            pltpu.SemaphoreType.DMA,
            pltpu.SemaphoreType.DMA,
            pltpu.SemaphoreType.DMA,
        ],
    )(functools.partial(_deg_body, ept=ept))


def _scatter_body(hp_hbm, ei_hbm, out_hbm,
                  row_v0, row_v1, row_v2, col_v0, col_v1, col_v2,
                  scol_v0, scol_v1, scol_v2, rows_v0, rows_v1, rows_v2,
                  acc_sh,
                  sem_i0, sem_i1, sem_i2, sem_g0, sem_g1, sem_g2,
                  sem_s0, sem_s1, sem_s2, *, ept):
    c = lax.axis_index("c")
    s = lax.axis_index("s")
    wid = c * NS + s
    ebase = wid * ept
    zero16 = jnp.zeros((16,), jnp.float32)

    row_v = [row_v0, row_v1, row_v2]
    col_v = [col_v0, col_v1, col_v2]
    scol_v = [scol_v0, scol_v1, scol_v2]
    rows_v = [rows_v0, rows_v1, rows_v2]
    sem_i = [sem_i0, sem_i1, sem_i2]
    sem_g = [sem_g0, sem_g1, sem_g2]
    sem_s = [sem_s0, sem_s1, sem_s2]

    def fillz(i, _):
        for j in range(D_HID // 16):
            rows_v0[i, pl.ds(j * 16, 16)] = zero16
        return 0
    lax.fori_loop(0, CHUNK, fillz, 0)

    pltpu.sync_copy(rows_v0, acc_sh.at[pl.ds(s * RPT, CHUNK)])
    pltpu.sync_copy(rows_v0.at[pl.ds(0, RPT - CHUNK)],
                    acc_sh.at[pl.ds(s * RPT + CHUNK, RPT - CHUNK)])

    nch = ept // CHUNK
    for k in range(3):
        pltpu.async_copy(ei_hbm.at[0, pl.ds(ebase + k * CHUNK, CHUNK)],
                         row_v[k], sem_i[k])
        pltpu.async_copy(ei_hbm.at[1, pl.ds(ebase + k * CHUNK, CHUNK)],
                         col_v[k], sem_i[k])
    plsc.subcore_barrier()

    def start_gather(g, k):
        @pl.when(g >= 3)
        def _():
            pltpu.make_async_copy(rows_v[k], acc_sh.at[scol_v[k]],
                                  sem_s[k]).wait()
        pltpu.make_async_copy(ei_hbm.at[0, pl.ds(ebase + g * CHUNK, CHUNK)],
                              row_v[k], sem_i[k]).wait()
        pltpu.make_async_copy(ei_hbm.at[1, pl.ds(ebase + g * CHUNK, CHUNK)],
                              col_v[k], sem_i[k]).wait()
        pltpu.async_copy(hp_hbm.at[row_v[k]], rows_v[k], sem_g[k])

    def finish_prev(g, k1):
        pltpu.make_async_copy(hp_hbm.at[row_v[k1]], rows_v[k1],
                              sem_g[k1]).wait()
        _copy_idx_local(col_v[k1], scol_v[k1], CHUNK)
        pltpu.make_async_copy(rows_v[k1], acc_sh.at[scol_v[k1]],
                              sem_s[k1]).start(add=True)
        @pl.when(g + 2 < nch)
        def _():
            nb = ebase + (g + 2) * CHUNK
            pltpu.async_copy(ei_hbm.at[0, pl.ds(nb, CHUNK)], row_v[k1],
                             sem_i[k1])
            pltpu.async_copy(ei_hbm.at[1, pl.ds(nb, CHUNK)], col_v[k1],
                             sem_i[k1])

    def body(g3, _):
        for k in range(3):
            g = g3 * 3 + k
            start_gather(g, k)
            @pl.when(g > 0)
            def _():
                finish_prev(g, (k + 2) % 3)
        return 0
    lax.fori_loop(0, nch // 3, body, 0)

    for t in range(nch - (nch // 3) * 3):
        g = (nch // 3) * 3 + t
        k = g % 3
        start_gather(g, k)
        finish_prev(g, (k + 2) % 3)
    klast = (nch - 1) % 3
    pltpu.make_async_copy(hp_hbm.at[row_v[klast]], rows_v[klast],
                          sem_g[klast]).wait()
    _copy_idx_local(col_v[klast], scol_v[klast], CHUNK)
    pltpu.make_async_copy(rows_v[klast], acc_sh.at[scol_v[klast]],
                          sem_s[klast]).start(add=True)

    for k in range(3):
        pltpu.make_async_copy(rows_v[k], acc_sh.at[scol_v[k]], sem_s[k]).wait()
    plsc.subcore_barrier()
    pltpu.sync_copy(acc_sh.at[pl.ds(s * RPT, RPT)],
                    out_hbm.at[c, pl.ds(s * RPT, RPT)])


def _make_scatter_call(E):
    ept = E // (NC * NS)
    return functools.partial(
        pl.kernel,
        mesh=_sc_mesh(),
        compiler_params=pltpu.CompilerParams(use_tc_tiling_on_sc=False),
        out_type=jax.ShapeDtypeStruct((NC, N_NODES, D_HID), jnp.float32),
        scratch_types=(
            [pltpu.VMEM((CHUNK,), jnp.int32)] * 6
            + [pltpu.VMEM((CHUNK,), jnp.int32)] * 3
            + [pltpu.VMEM((CHUNK, D_HID), jnp.float32)] * 3
            + [pltpu.VMEM_SHARED((N_NODES, D_HID), jnp.float32)]
            + [pltpu.SemaphoreType.DMA] * 9
        ),
    )(functools.partial(_scatter_body, ept=ept))


def _dinv_block(dp_ref):
    deg = dp_ref[0, :, 0:1] + dp_ref[1, :, 0:1] + 1.0
    return lax.rsqrt(deg)


def _dotT(a, w):
    return lax.dot_general(a, w, (((1,), (1,)), ((), ())),
                           preferred_element_type=jnp.float32,
                           precision=lax.Precision.HIGHEST)


def _tc1_body(dp_ref, x_ref, w_ref, hp_ref):
    hp_ref[...] = _dinv_block(dp_ref) * _dotT(x_ref[...], w_ref[...])


def _tc1_call(degp, x, w1):
    n, din = x.shape
    grid = n // ROWS_TC
    return pl.pallas_call(
        _tc1_body,
        grid=(grid,),
        in_specs=[
            pl.BlockSpec((NC, ROWS_TC, 16), lambda i: (0, i, 0)),
            pl.BlockSpec((ROWS_TC, din), lambda i: (i, 0)),
            pl.BlockSpec((D_HID, din), lambda i: (0, 0)),
        ],
        out_specs=pl.BlockSpec((ROWS_TC, D_HID), lambda i: (i, 0)),
        out_shape=jax.ShapeDtypeStruct((n, D_HID), jnp.float32),
    )(degp, x, w1)


def _tc_mid_body(acc_ref, dp_ref, hp_ref, b_ref, w_ref, out_ref):
    dinv = _dinv_block(dp_ref)
    agg = dinv * (acc_ref[0] + acc_ref[1] + hp_ref[...]) + b_ref[...]
    h = jnp.where(agg >= 0, agg, 0.01 * agg)
    out_ref[...] = dinv * _dotT(h, w_ref[...])


def _tc_mid_call(acc, degp, hp, b, w):
    n = hp.shape[0]
    grid = n // ROWS_TC
    return pl.pallas_call(
        _tc_mid_body,
        grid=(grid,),
        in_specs=[
            pl.BlockSpec((NC, ROWS_TC, D_HID), lambda i: (0, i, 0)),
            pl.BlockSpec((NC, ROWS_TC, 16), lambda i: (0, i, 0)),
            pl.BlockSpec((ROWS_TC, D_HID), lambda i: (i, 0)),
            pl.BlockSpec((1, D_HID), lambda i: (0, 0)),
            pl.BlockSpec((D_HID, D_HID), lambda i: (0, 0)),
        ],
        out_specs=pl.BlockSpec((ROWS_TC, D_HID), lambda i: (i, 0)),
        out_shape=jax.ShapeDtypeStruct((n, D_HID), jnp.float32),
    )(acc, degp, hp, b, w)


def _tc_fin_body(acc_ref, dp_ref, hp_ref, b_ref, out_ref):
    dinv = _dinv_block(dp_ref)
    agg = dinv * (acc_ref[0] + acc_ref[1] + hp_ref[...]) + b_ref[...]
    out_ref[...] = jnp.where(agg >= 0, agg, 0.01 * agg)


def _tc_fin_call(acc, degp, hp, b):
    n = hp.shape[0]
    grid = n // ROWS_TC
    return pl.pallas_call(
        _tc_fin_body,
        grid=(grid,),
        in_specs=[
            pl.BlockSpec((NC, ROWS_TC, D_HID), lambda i: (0, i, 0)),
            pl.BlockSpec((NC, ROWS_TC, 16), lambda i: (0, i, 0)),
            pl.BlockSpec((ROWS_TC, D_HID), lambda i: (i, 0)),
            pl.BlockSpec((1, D_HID), lambda i: (0, 0)),
        ],
        out_specs=pl.BlockSpec((ROWS_TC, D_HID), lambda i: (i, 0)),
        out_shape=jax.ShapeDtypeStruct((n, D_HID), jnp.float32),
    )(acc, degp, hp, b)


def kernel(x, edge_index, batch, W1, b1, W2, b2, W3, b3):
    E = edge_index.shape[1]

    degp = _make_deg_call(E)(edge_index)
    hp1 = _tc1_call(degp, x, W1)

    scat = _make_scatter_call(E)
    acc = scat(hp1, edge_index)
    hp2 = _tc_mid_call(acc, degp, hp1, b1.reshape(1, -1), W2)
    acc = scat(hp2, edge_index)
    hp3 = _tc_mid_call(acc, degp, hp2, b2.reshape(1, -1), W3)
    acc = scat(hp3, edge_index)
    return _tc_fin_call(acc, degp, hp3, b3.reshape(1, -1))
